# baseline (speedup 1.0000x reference)
import jax
import jax.numpy as jnp
from jax.experimental import pallas as pl
from jax.experimental.pallas import tpu as pltpu
from jax.experimental.pallas import tpu_sc as plsc


_LANES = 16
_BMR = 4


def _sc_kernel(x, pe):
    B, S, D = x.shape

    mesh = plsc.VectorSubcoreMesh(core_axis_name="core",
                                  subcore_axis_name="subcore")

    @pl.kernel(out_type=jax.ShapeDtypeStruct((B, S, D), x.dtype),
               mesh=mesh, scratch_types=[])
    def sc_add(x_hbm, pe_hbm, o_hbm):
        def body(x_vmem, pe_vmem, o_vmem):
            @pl.loop(0, _BMR)
            def _(r):
                @plsc.parallel_loop(0, D, step=_LANES, unroll=16)
                def _(c):
                    sl = (pl.ds(r, 1), pl.ds(c, _LANES))
                    p = pe_vmem.at[*sl][...]
                    for b in range(B):
                        o_vmem.at[b, *sl][...] = x_vmem.at[b, *sl][...] + p

        pltpu.emit_pipeline(
            body,
            grid=(S // _BMR,),
            in_specs=[
                pl.BlockSpec((B, _BMR, D), index_map=lambda i: (0, i, 0)),
                pl.BlockSpec((_BMR, D), index_map=lambda i: (i, 0)),
            ],
            out_specs=[pl.BlockSpec((B, _BMR, D), index_map=lambda i: (0, i, 0))],
            core_axis_name=("core", "subcore"),
            dimension_semantics=(pltpu.PARALLEL,),
        )(x_hbm, pe_hbm, o_hbm)

    return sc_add(x, pe)


def kernel(x, pe_weight):
    B, S, D = x.shape
    return _sc_kernel(x, pe_weight[:S])


# speedup vs baseline: 1.0036x; 1.0036x over previous
import jax
import jax.numpy as jnp
from jax.experimental import pallas as pl
from jax.experimental.pallas import tpu as pltpu
from jax.experimental.pallas import tpu_sc as plsc


_LANES = 16
_BMR = 4


def _sc_kernel(x, pe):
    B, S, D = x.shape

    mesh = plsc.VectorSubcoreMesh(core_axis_name="core",
                                  subcore_axis_name="subcore")

    @pl.kernel(out_type=jax.ShapeDtypeStruct((B, S, D), x.dtype),
               mesh=mesh, scratch_types=[])
    def sc_add(x_hbm, pe_hbm, o_hbm):
        def body(x_vmem, pe_vmem, o_vmem):
            @pl.loop(0, _BMR)
            def _(r):
                @plsc.parallel_loop(0, D, step=_LANES, unroll=8)
                def _(c):
                    sl = (pl.ds(r, 1), pl.ds(c, _LANES))
                    p = pe_vmem.at[*sl][...]
                    for b in range(B):
                        o_vmem.at[b, *sl][...] = x_vmem.at[b, *sl][...] + p

        pltpu.emit_pipeline(
            body,
            grid=(S // _BMR,),
            in_specs=[
                pl.BlockSpec((B, _BMR, D), index_map=lambda i: (0, i, 0)),
                pl.BlockSpec((_BMR, D), index_map=lambda i: (i, 0)),
            ],
            out_specs=[pl.BlockSpec((B, _BMR, D), index_map=lambda i: (0, i, 0))],
            core_axis_name=("core", "subcore"),
            dimension_semantics=(pltpu.PARALLEL,),
        )(x_hbm, pe_hbm, o_hbm)

    return sc_add(x, pe)


def kernel(x, pe_weight):
    B, S, D = x.shape
    return _sc_kernel(x, pe_weight[:S])
